# Initial kernel scaffold; baseline (speedup 1.0000x reference)
import jax
import jax.numpy as jnp
from jax.experimental import pallas as pl


def kernel(node_attr, edge_index, edge_attr, global_attr, W1, b1, W2, b2):
    raise NotImplementedError("write your pallas kernel here")


# speedup vs baseline: 6.4428x; 6.4428x over previous
import functools

import jax
import jax.numpy as jnp
from jax.experimental import pallas as pl
from jax.experimental.pallas import tpu as pltpu

_GRID = 10


def _body(a_ref, b_ref, g_ref, wg_ref, we_ref, wn_ref, b1_ref, w2_ref,
          b2_ref, o_ref, acc_ref, *, grid, inv_e, inv_n):
    i = pl.program_id(0)
    ea = jnp.sum(a_ref[...], axis=0, keepdims=True)
    na = jnp.sum(b_ref[...], axis=0, keepdims=True)

    @pl.when(i == 0)
    def _init():
        acc_ref[0:1, :] = ea
        acc_ref[1:2, :] = na

    @pl.when(i > 0)
    def _acc():
        acc_ref[0:1, :] = acc_ref[0:1, :] + ea
        acc_ref[1:2, :] = acc_ref[1:2, :] + na

    @pl.when(i == grid - 1)
    def _finish():
        emean = acc_ref[0:1, :] * inv_e
        nmean = acc_ref[1:2, :] * inv_n
        pre = (g_ref[...] @ wg_ref[...] + emean @ we_ref[...]
               + nmean @ wn_ref[...] + b1_ref[...])
        h = jnp.maximum(pre, 0.0)
        o_ref[...] = h @ w2_ref[...] + b2_ref[...]


def kernel(node_attr, edge_index, edge_attr, global_attr, W1, b1, W2, b2):
    del edge_index
    n_edges, d_edge = edge_attr.shape
    n_nodes, d_feat = node_attr.shape
    d_global = global_attr.shape[1]
    fold = 128 // d_edge
    a = edge_attr.reshape(n_edges // fold, d_edge * fold)
    wg = W1[:d_global]
    we = jnp.tile(W1[d_global:d_global + d_edge], (fold, 1))
    wn = W1[d_global + d_edge:]

    grid = _GRID
    blk_a = a.shape[0] // grid
    blk_b = n_nodes // grid

    body = functools.partial(_body, grid=grid,
                             inv_e=1.0 / n_edges, inv_n=1.0 / n_nodes)
    out = pl.pallas_call(
        body,
        grid=(grid,),
        in_specs=[
            pl.BlockSpec((blk_a, 128), lambda i: (i, 0)),
            pl.BlockSpec((blk_b, d_feat), lambda i: (i, 0)),
            pl.BlockSpec((1, d_global), lambda i: (0, 0)),
            pl.BlockSpec((d_global, 32), lambda i: (0, 0)),
            pl.BlockSpec((128, 32), lambda i: (0, 0)),
            pl.BlockSpec((d_feat, 32), lambda i: (0, 0)),
            pl.BlockSpec((1, 32), lambda i: (0, 0)),
            pl.BlockSpec((32, 128), lambda i: (0, 0)),
            pl.BlockSpec((1, 128), lambda i: (0, 0)),
        ],
        out_specs=pl.BlockSpec((1, 128), lambda i: (0, 0)),
        out_shape=jax.ShapeDtypeStruct((1, 128), jnp.float32),
        scratch_shapes=[pltpu.VMEM((8, 128), jnp.float32)],
    )(a, node_attr, global_attr, wg, we, wn,
      b1.reshape(1, 32), W2, b2.reshape(1, 128))
    return out


# speedup vs baseline: 7.0465x; 1.0937x over previous
import functools

import jax
import jax.numpy as jnp
from jax.experimental import pallas as pl
from jax.experimental.pallas import tpu as pltpu

_GRID = 10


def _body(a_ref, b_ref, g_ref, w1_ref, b1_ref, w2_ref, b2_ref,
          o_ref, acc_ref, *, grid, inv_e, inv_n, d_edge, d_global):
    i = pl.program_id(0)
    ea = jnp.sum(a_ref[...], axis=0, keepdims=True)
    na = jnp.sum(b_ref[...], axis=0, keepdims=True)

    @pl.when(i == 0)
    def _init():
        acc_ref[0:1, :d_edge] = ea
        acc_ref[1:2, :] = na

    @pl.when(i > 0)
    def _acc():
        acc_ref[0:1, :d_edge] = acc_ref[0:1, :d_edge] + ea
        acc_ref[1:2, :] = acc_ref[1:2, :] + na

    @pl.when(i == grid - 1)
    def _finish():
        emean = acc_ref[0:1, :d_edge] * inv_e
        nmean = acc_ref[1:2, :] * inv_n
        wg = w1_ref[:d_global, :]
        we = w1_ref[d_global:d_global + d_edge, :]
        wn = w1_ref[d_global + d_edge:, :]
        pre = (g_ref[...] @ wg + emean @ we + nmean @ wn
               + b1_ref[...][None, :])
        h = jnp.maximum(pre, 0.0)
        o_ref[...] = h @ w2_ref[...] + b2_ref[...][None, :]


def kernel(node_attr, edge_index, edge_attr, global_attr, W1, b1, W2, b2):
    del edge_index
    n_edges, d_edge = edge_attr.shape
    n_nodes, d_feat = node_attr.shape
    d_global = global_attr.shape[1]
    in_features, latent = W1.shape
    out_features = W2.shape[1]

    grid = _GRID
    blk_a = n_edges // grid
    blk_b = n_nodes // grid

    body = functools.partial(_body, grid=grid, inv_e=1.0 / n_edges,
                             inv_n=1.0 / n_nodes, d_edge=d_edge,
                             d_global=d_global)
    out = pl.pallas_call(
        body,
        grid=(grid,),
        in_specs=[
            pl.BlockSpec((blk_a, d_edge), lambda i: (i, 0)),
            pl.BlockSpec((blk_b, d_feat), lambda i: (i, 0)),
            pl.BlockSpec((1, d_global), lambda i: (0, 0)),
            pl.BlockSpec((in_features, latent), lambda i: (0, 0)),
            pl.BlockSpec((latent,), lambda i: (0,)),
            pl.BlockSpec((latent, out_features), lambda i: (0, 0)),
            pl.BlockSpec((out_features,), lambda i: (0,)),
        ],
        out_specs=pl.BlockSpec((1, out_features), lambda i: (0, 0)),
        out_shape=jax.ShapeDtypeStruct((1, out_features), jnp.float32),
        scratch_shapes=[pltpu.VMEM((8, d_feat), jnp.float32)],
    )(edge_attr, node_attr, global_attr, W1, b1, W2, b2)
    return out


# speedup vs baseline: 7.2445x; 1.0281x over previous
import functools

import jax
import jax.numpy as jnp
from jax.experimental import pallas as pl
from jax.experimental.pallas import tpu as pltpu

_GRID = 10
_NA = 8
_NB = 5


def _body(*refs, grid, inv_e, inv_n, d_edge, d_global):
    a_refs = refs[:_NA]
    b_refs = refs[_NA:_NA + _NB]
    g_ref, w1_ref, b1_ref, w2_ref, b2_ref, o_ref, acc_ref = refs[_NA + _NB:]
    i = pl.program_id(0)

    ea = a_refs[0][...].sum(axis=0, keepdims=True)
    for r in a_refs[1:]:
        ea = ea + r[...].sum(axis=0, keepdims=True)
    na = b_refs[0][...].sum(axis=0, keepdims=True)
    for r in b_refs[1:]:
        na = na + r[...].sum(axis=0, keepdims=True)

    @pl.when(i == 0)
    def _init():
        acc_ref[0:1, :d_edge] = ea
        acc_ref[1:2, :] = na

    @pl.when(i > 0)
    def _acc():
        acc_ref[0:1, :d_edge] = acc_ref[0:1, :d_edge] + ea
        acc_ref[1:2, :] = acc_ref[1:2, :] + na

    @pl.when(i == grid - 1)
    def _finish():
        emean = acc_ref[0:1, :d_edge] * inv_e
        nmean = acc_ref[1:2, :] * inv_n
        wg = w1_ref[:d_global, :]
        we = w1_ref[d_global:d_global + d_edge, :]
        wn = w1_ref[d_global + d_edge:, :]
        pre = (g_ref[...] @ wg + emean @ we + nmean @ wn
               + b1_ref[...][None, :])
        h = jnp.maximum(pre, 0.0)
        o_ref[...] = h @ w2_ref[...] + b2_ref[...][None, :]


def kernel(node_attr, edge_index, edge_attr, global_attr, W1, b1, W2, b2):
    del edge_index
    n_edges, d_edge = edge_attr.shape
    n_nodes, d_feat = node_attr.shape
    d_global = global_attr.shape[1]
    in_features, latent = W1.shape
    out_features = W2.shape[1]

    grid = _GRID
    blk_a = n_edges // (grid * _NA)
    blk_b = n_nodes // (grid * _NB)

    def a_spec(j):
        return pl.BlockSpec((blk_a, d_edge), lambda i, j=j: (j * grid + i, 0))

    def b_spec(j):
        return pl.BlockSpec((blk_b, d_feat), lambda i, j=j: (j * grid + i, 0))

    body = functools.partial(_body, grid=grid, inv_e=1.0 / n_edges,
                             inv_n=1.0 / n_nodes, d_edge=d_edge,
                             d_global=d_global)
    out = pl.pallas_call(
        body,
        grid=(grid,),
        in_specs=(
            [a_spec(j) for j in range(_NA)]
            + [b_spec(j) for j in range(_NB)]
            + [
                pl.BlockSpec((1, d_global), lambda i: (0, 0)),
                pl.BlockSpec((in_features, latent), lambda i: (0, 0)),
                pl.BlockSpec((latent,), lambda i: (0,)),
                pl.BlockSpec((latent, out_features), lambda i: (0, 0)),
                pl.BlockSpec((out_features,), lambda i: (0,)),
            ]
        ),
        out_specs=pl.BlockSpec((1, out_features), lambda i: (0, 0)),
        out_shape=jax.ShapeDtypeStruct((1, out_features), jnp.float32),
        scratch_shapes=[pltpu.VMEM((8, d_feat), jnp.float32)],
    )(*([edge_attr] * _NA), *([node_attr] * _NB),
      global_attr, W1, b1, W2, b2)
    return out
